# Initial kernel scaffold; baseline (speedup 1.0000x reference)
import jax
import jax.numpy as jnp
from jax.experimental import pallas as pl


def kernel(classification, labels):
    raise NotImplementedError("write your pallas kernel here")


# speedup vs baseline: 2.6378x; 2.6378x over previous
import jax
import jax.numpy as jnp
from jax import lax
from jax.experimental import pallas as pl
from jax.experimental.pallas import tpu as pltpu
from jax.experimental.pallas import tpu_sc as plsc

NUM_CLASSES = 21
ALPHA = 0.75
N_ROWS = 16 * 100000
NC, NS, LANES = 2, 16, 16
NW = NC * NS
PER_W = N_ROWS // NW
CHUNK = 10_000
NCHUNK = PER_W // CHUNK
VECS = CHUNK // LANES


def _focal_body(c_hbm, l_hbm, out_hbm, lbl_v, idx_v, prob_v, out_v, sem):
    wid = lax.axis_index("s") * NC + lax.axis_index("c")
    base = wid * PER_W
    iota21 = lax.iota(jnp.int32, LANES) * NUM_CLASSES

    def chunk_body(k, carry):
        start = base + k * CHUNK
        pltpu.sync_copy(l_hbm.at[pl.ds(start, CHUNK)], lbl_v)
        row0_t21 = start * NUM_CLASSES

        def idx_body(o, c2):
            off = o * LANES
            lbl = lbl_v[pl.ds(off, LANES)]
            idx_v[pl.ds(off, LANES)] = (row0_t21 + off * NUM_CLASSES) + iota21 + lbl
            return c2

        lax.fori_loop(0, VECS, idx_body, 0, unroll=4)

        pltpu.async_copy(c_hbm.at[idx_v], prob_v, sem).wait()

        def cmp_body(o, c2):
            off = o * LANES
            lbl = lbl_v[pl.ds(off, LANES)]
            p = prob_v[pl.ds(off, LANES)]
            a = jnp.where(lbl == 0, 1.0 - ALPHA, ALPHA).astype(jnp.float32)
            r = 1.0 - p
            out_v[pl.ds(off, LANES)] = a * r * r
            return c2

        lax.fori_loop(0, VECS, cmp_body, 0, unroll=4)
        pltpu.sync_copy(out_v, out_hbm.at[pl.ds(start, CHUNK)])
        return carry

    lax.fori_loop(0, NCHUNK, chunk_body, 0)


_focal_call = pl.kernel(
    _focal_body,
    out_type=jax.ShapeDtypeStruct((N_ROWS,), jnp.float32),
    mesh=plsc.VectorSubcoreMesh(core_axis_name="c", subcore_axis_name="s"),
    scratch_types=[
        pltpu.VMEM((CHUNK,), jnp.int32),
        pltpu.VMEM((CHUNK,), jnp.int32),
        pltpu.VMEM((CHUNK,), jnp.float32),
        pltpu.VMEM((CHUNK,), jnp.float32),
        pltpu.SemaphoreType.DMA,
    ],
)


def kernel(classification, labels):
    cflat = classification.reshape(-1)
    lflat = labels.reshape(-1).astype(jnp.int32)
    return _focal_call(cflat, lflat)


# speedup vs baseline: 20.3993x; 7.7334x over previous
import jax
import jax.numpy as jnp
from jax import lax
from jax.experimental import pallas as pl
from jax.experimental.pallas import tpu as pltpu
from jax.experimental.pallas import tpu_sc as plsc

NUM_CLASSES = 21
ALPHA = 0.75
B, R = 16, 100000
NC, NS, LANES = 2, 16, 16
NW = NC * NS
W = 128
NBLK_PER_S = 99968 // W
R_IN = NBLK_PER_S * W
NBLK = 2 * NBLK_PER_S
GRPS = W // LANES


def _focal_body(ct_hbm, lbl_hbm, out_hbm, class_v, lbl_v, out_v):
    wid = lax.axis_index("s") * NC + lax.axis_index("c")
    iota16 = lax.iota(jnp.int32, LANES)
    nblk = jnp.where(wid < NBLK - NW * (NBLK // NW), NBLK // NW + 1, NBLK // NW)

    def blk(t, carry):
        g = wid + NW * t
        stile = jnp.where(g >= NBLK_PER_S, 1, 0)
        r0 = pl.multiple_of((g - stile * NBLK_PER_S) * W, W)
        b0 = pl.multiple_of(stile * 8, 8)
        pltpu.sync_copy(ct_hbm.at[:, pl.ds(b0, 8), pl.ds(r0, W)], class_v)
        pltpu.sync_copy(lbl_hbm.at[pl.ds(b0, 8), pl.ds(r0, W)], lbl_v)
        for s in range(8):

            def grp(i, c2, s=s):
                c0 = i * LANES
                lbl = lbl_v[s, pl.ds(c0, LANES)]
                p = class_v[0, s, pl.ds(c0, LANES)]
                for k in range(1, NUM_CLASSES):
                    p = jnp.where(lbl == k, class_v[k, s, pl.ds(c0, LANES)], p)
                a = jnp.where(lbl == 0, 1.0 - ALPHA, ALPHA).astype(jnp.float32)
                r = 1.0 - p
                out_v[s, pl.ds(c0, LANES)] = (a * r) * r
                return c2

            lax.fori_loop(0, GRPS, grp, 0, unroll=2)
        pltpu.sync_copy(out_v, out_hbm.at[pl.ds(b0, 8), pl.ds(r0, W)])
        return carry

    lax.fori_loop(0, nblk, blk, 0)


_focal_call = pl.kernel(
    _focal_body,
    out_type=jax.ShapeDtypeStruct((B, R_IN), jnp.float32),
    mesh=plsc.VectorSubcoreMesh(core_axis_name="c", subcore_axis_name="s"),
    scratch_types=[
        pltpu.VMEM((NUM_CLASSES, 8, W), jnp.float32),
        pltpu.VMEM((8, W), jnp.int32),
        pltpu.VMEM((8, W), jnp.float32),
    ],
)


def kernel(classification, labels):
    lbl = labels.astype(jnp.int32)
    ct = jnp.transpose(classification, (2, 0, 1))
    out2d = _focal_call(ct, lbl)
    lbl_t = lbl[:, R_IN:]
    p_t = jnp.take_along_axis(classification[:, R_IN:, :], lbl_t[:, :, None], axis=2)[:, :, 0]
    a_t = jnp.where(lbl_t == 0, 1.0 - ALPHA, ALPHA).astype(jnp.float32)
    tail = a_t * (1.0 - p_t) ** 2
    return jnp.concatenate([out2d, tail], axis=1).reshape(-1)


# speedup vs baseline: 31.8230x; 1.5600x over previous
import jax
import jax.numpy as jnp
from jax import lax
from jax.experimental import pallas as pl
from jax.experimental.pallas import tpu as pltpu
from jax.experimental.pallas import tpu_sc as plsc

NUM_CLASSES = 21
ALPHA = 0.75
B, R = 16, 100000
NC, NS, LANES = 2, 16, 16
NW = NC * NS
W = 128
NBLK_PER_S = 99968 // W
R_IN = NBLK_PER_S * W
NBLK = 2 * NBLK_PER_S
GRPS = W // LANES
NT = -(-NBLK // NW)
NT_EVEN = NT + (NT % 2)


def _focal_body(ct_hbm, lbl_hbm, out_hbm, class_v, lbl_v, out_v, in_sems, out_sems):
    wid = lax.axis_index("s") * NC + lax.axis_index("c")

    def slab_coords(t):
        g = jnp.minimum(wid + NW * t, NBLK - 1)
        stile = jnp.where(g >= NBLK_PER_S, 1, 0)
        r0 = pl.multiple_of((g - stile * NBLK_PER_S) * W, W)
        b0 = pl.multiple_of(stile * 8, 8)
        return b0, r0

    def start_in(b, t):
        b0, r0 = slab_coords(t)
        pltpu.async_copy(
            ct_hbm.at[:, pl.ds(b0, 8), pl.ds(r0, W)], class_v.at[b], in_sems.at[b]
        )
        pltpu.async_copy(
            lbl_hbm.at[pl.ds(b0, 8), pl.ds(r0, W)], lbl_v.at[b], in_sems.at[b]
        )

    def wait_in(b):
        pltpu.make_async_copy(
            ct_hbm.at[:, pl.ds(0, 8), pl.ds(0, W)], class_v.at[b], in_sems.at[b]
        ).wait()
        pltpu.make_async_copy(
            lbl_hbm.at[pl.ds(0, 8), pl.ds(0, W)], lbl_v.at[b], in_sems.at[b]
        ).wait()

    def start_out(b, t):
        b0, r0 = slab_coords(t)
        pltpu.async_copy(
            out_v.at[b], out_hbm.at[pl.ds(b0, 8), pl.ds(r0, W)], out_sems.at[b]
        )

    def wait_out(b):
        pltpu.make_async_copy(
            out_v.at[b], out_hbm.at[pl.ds(0, 8), pl.ds(0, W)], out_sems.at[b]
        ).wait()

    def compute(b):
        for s in range(8):

            def grp(i, c2, s=s):
                c0 = i * LANES
                lbl = lbl_v[b, s, pl.ds(c0, LANES)]
                p = class_v[b, 0, s, pl.ds(c0, LANES)]
                for k in range(1, NUM_CLASSES):
                    p = jnp.where(lbl == k, class_v[b, k, s, pl.ds(c0, LANES)], p)
                a = jnp.where(lbl == 0, 1.0 - ALPHA, ALPHA).astype(jnp.float32)
                r = 1.0 - p
                out_v[b, s, pl.ds(c0, LANES)] = (a * r) * r
                return c2

            lax.fori_loop(0, GRPS, grp, 0, unroll=2)

    start_in(0, 0)

    def pair(tp, carry):
        t0 = 2 * tp
        for par in range(2):
            t = t0 + par
            nxt = 1 - par

            @pl.when(t + 1 < NT_EVEN)
            def _():
                start_in(nxt, t + 1)

            wait_in(par)

            @pl.when(tp > 0)
            def _():
                wait_out(par)

            compute(par)
            start_out(par, t)
        return carry

    lax.fori_loop(0, NT_EVEN // 2, pair, 0)
    wait_out(0)
    wait_out(1)


_focal_call = pl.kernel(
    _focal_body,
    out_type=jax.ShapeDtypeStruct((B, R_IN), jnp.float32),
    mesh=plsc.VectorSubcoreMesh(core_axis_name="c", subcore_axis_name="s"),
    scratch_types=[
        pltpu.VMEM((2, NUM_CLASSES, 8, W), jnp.float32),
        pltpu.VMEM((2, 8, W), jnp.int32),
        pltpu.VMEM((2, 8, W), jnp.float32),
        pltpu.SemaphoreType.DMA((2,)),
        pltpu.SemaphoreType.DMA((2,)),
    ],
)


def kernel(classification, labels):
    lbl = labels.astype(jnp.int32)
    ct = jnp.transpose(classification, (2, 0, 1))
    out2d = _focal_call(ct, lbl)
    lbl_t = lbl[:, R_IN:]
    p_t = jnp.take_along_axis(classification[:, R_IN:, :], lbl_t[:, :, None], axis=2)[:, :, 0]
    a_t = jnp.where(lbl_t == 0, 1.0 - ALPHA, ALPHA).astype(jnp.float32)
    tail = a_t * (1.0 - p_t) ** 2
    return jnp.concatenate([out2d, tail], axis=1).reshape(-1)
